# initial kernel scaffold (unmeasured)
import functools

import jax
import jax.numpy as jnp
from jax import lax
from jax.experimental import pallas as pl
from jax.experimental.pallas import tpu as pltpu

N_DEV = 8
B = 2
SQ = 512
SKV = 512
HQ_LOC = 8
DH = 64
D_MODEL = 768
D_HID_LOC = HQ_LOC * DH
WINDOW = 128
ROWS = B * SQ
CHUNK = ROWS // N_DEV


def _mod(a):
    return lax.rem(a + N_DEV, N_DEV)


def kernel(x, Wq, K_ext, V_ext, Wo):
    x2 = x.reshape(ROWS, D_MODEL)

    def body(x_ref, wq_ref, k_ref, v_ref, wo_ref, out_ref,
             wq_loc, wo_loc, partial_ref, rs_buf, ag_buf,
             copy_sems, send_sems, recv_sems):
        my = lax.axis_index("i")
        left = _mod(my - 1)
        right = _mod(my + 1)

        wq_copy = pltpu.make_async_copy(
            wq_ref.at[:, pl.ds(my * D_HID_LOC, D_HID_LOC)],
            wq_loc, copy_sems.at[0])
        wq_copy.start()
        wo_copy = pltpu.make_async_copy(
            wo_ref.at[pl.ds(my * D_HID_LOC, D_HID_LOC), :],
            wo_loc, copy_sems.at[1])
        wo_copy.start()

        barrier_sem = pltpu.get_barrier_semaphore()
        for nbr in (left, right):
            pl.semaphore_signal(barrier_sem, inc=1, device_id=(nbr,),
                                device_id_type=pl.DeviceIdType.MESH)
        pl.semaphore_wait(barrier_sem, 2)

        wq_copy.wait()
        wo_copy.wait()

        iq = lax.broadcasted_iota(jnp.int32, (SQ, SKV), 0)
        ik = lax.broadcasted_iota(jnp.int32, (SQ, SKV), 1)
        band = jnp.abs(iq - ik) <= WINDOW

        wq_bf = wq_loc[:, :].astype(jnp.bfloat16)
        wo_bf = wo_loc[:, :].astype(jnp.bfloat16)
        x_bf = x_ref[:, :].astype(jnp.bfloat16)
        q_all = jax.lax.dot(x_bf, wq_bf,
                            preferred_element_type=jnp.float32)

        for b in range(B):
            ctx_cols = []
            for h in range(HQ_LOC):
                qh = q_all[b * SQ:(b + 1) * SQ,
                           h * DH:(h + 1) * DH].astype(jnp.bfloat16)
                kh = k_ref[b, :, h, :].astype(jnp.bfloat16)
                scores = jax.lax.dot_general(
                    qh, kh, (((1,), (1,)), ((), ())),
                    preferred_element_type=jnp.float32) * 0.125
                scores = jnp.where(band, scores, jnp.float32(-1e9))
                m = jnp.max(scores, axis=1, keepdims=True)
                w = jnp.exp(scores - m)
                w = w / jnp.sum(w, axis=1, keepdims=True)
                vh = v_ref[b, :, h, :].astype(jnp.bfloat16)
                ctx_cols.append(jax.lax.dot(
                    w.astype(jnp.bfloat16), vh,
                    preferred_element_type=jnp.float32))
            ctx_b = jnp.concatenate(ctx_cols, axis=1)
            partial_ref[b * SQ:(b + 1) * SQ, :] = jax.lax.dot(
                ctx_b.astype(jnp.bfloat16), wo_bf,
                preferred_element_type=jnp.float32)

        for s in range(N_DEV - 1):
            send_c = _mod(my - s)
            if s == 0:
                src = partial_ref.at[pl.ds(send_c * CHUNK, CHUNK), :]
            else:
                src = rs_buf.at[s - 1]
            rdma = pltpu.make_async_remote_copy(
                src_ref=src,
                dst_ref=rs_buf.at[s],
                send_sem=send_sems.at[s],
                recv_sem=recv_sems.at[s],
                device_id=(right,),
                device_id_type=pl.DeviceIdType.MESH,
            )
            rdma.start()
            rdma.wait()
            recv_c = _mod(my - s - 1)
            rs_buf[s, :, :] = (
                rs_buf[s, :, :]
                + partial_ref[pl.ds(recv_c * CHUNK, CHUNK), :])

        my_c = _mod(my + 1)
        out_ref[pl.ds(my_c * CHUNK, CHUNK), :] = rs_buf[N_DEV - 2, :, :]

        for s in range(N_DEV - 1):
            if s == 0:
                src = rs_buf.at[N_DEV - 2]
            else:
                src = ag_buf.at[s - 1]
            rdma = pltpu.make_async_remote_copy(
                src_ref=src,
                dst_ref=ag_buf.at[s],
                send_sem=send_sems.at[N_DEV - 1 + s],
                recv_sem=recv_sems.at[N_DEV - 1 + s],
                device_id=(right,),
                device_id_type=pl.DeviceIdType.MESH,
            )
            rdma.start()
            rdma.wait()
            c = _mod(my - s)
            out_ref[pl.ds(c * CHUNK, CHUNK), :] = ag_buf[s, :, :]

        @functools.partial(pl.run_scoped,
                           second_barrier=pltpu.SemaphoreType.REGULAR)
        def _(second_barrier):
            for nbr in (left, right):
                pl.semaphore_signal(second_barrier, inc=1, device_id=(nbr,),
                                    device_id_type=pl.DeviceIdType.MESH)
            pl.semaphore_wait(second_barrier, 2)

    out = pl.pallas_call(
        body,
        out_shape=jax.ShapeDtypeStruct((ROWS, D_MODEL), jnp.float32),
        in_specs=[
            pl.BlockSpec(memory_space=pltpu.VMEM),
            pl.BlockSpec(memory_space=pltpu.ANY),
            pl.BlockSpec(memory_space=pltpu.VMEM),
            pl.BlockSpec(memory_space=pltpu.VMEM),
            pl.BlockSpec(memory_space=pltpu.ANY),
        ],
        out_specs=pl.BlockSpec(memory_space=pltpu.VMEM),
        scratch_shapes=[
            pltpu.VMEM((D_MODEL, D_HID_LOC), jnp.float32),
            pltpu.VMEM((D_HID_LOC, D_MODEL), jnp.float32),
            pltpu.VMEM((ROWS, D_MODEL), jnp.float32),
            pltpu.VMEM((N_DEV - 1, CHUNK, D_MODEL), jnp.float32),
            pltpu.VMEM((N_DEV - 1, CHUNK, D_MODEL), jnp.float32),
            pltpu.SemaphoreType.DMA((2,)),
            pltpu.SemaphoreType.DMA((2 * (N_DEV - 1),)),
            pltpu.SemaphoreType.DMA((2 * (N_DEV - 1),)),
        ],
        compiler_params=pltpu.CompilerParams(collective_id=0),
    )(x2, Wq, K_ext, V_ext, Wo)

    return out.reshape(B, SQ, D_MODEL)


# baseline (device time: 113886 ns/iter reference)
import functools

import jax
import jax.numpy as jnp
from jax import lax
from jax.experimental import pallas as pl
from jax.experimental.pallas import tpu as pltpu

N_DEV = 8
B = 2
SQ = 512
SKV = 512
HQ_LOC = 8
DH = 64
D_MODEL = 768
D_HID_LOC = HQ_LOC * DH
WINDOW = 128
ROWS = B * SQ
CHUNK = ROWS // N_DEV


def _mod(a):
    return lax.rem(a + N_DEV, N_DEV)


def kernel(x, Wq, K_ext, V_ext, Wo):
    x2 = x.reshape(ROWS, D_MODEL)

    def body(x_ref, wq_ref, k_ref, v_ref, wo_ref, out_ref,
             wq_loc, wo_loc, partial_ref, rs_buf, ag_buf,
             copy_sems, send_sems, recv_sems):
        my = lax.axis_index("i")
        left = _mod(my - 1)
        right = _mod(my + 1)

        wq_copy = pltpu.make_async_copy(
            wq_ref.at[:, pl.ds(my * D_HID_LOC, D_HID_LOC)],
            wq_loc, copy_sems.at[0])
        wq_copy.start()
        wo_copy = pltpu.make_async_copy(
            wo_ref.at[pl.ds(my * D_HID_LOC, D_HID_LOC), :],
            wo_loc, copy_sems.at[1])
        wo_copy.start()

        barrier_sem = pltpu.get_barrier_semaphore()
        for nbr in (left, right):
            pl.semaphore_signal(barrier_sem, inc=1, device_id=(nbr,),
                                device_id_type=pl.DeviceIdType.MESH)
        pl.semaphore_wait(barrier_sem, 2)

        wq_copy.wait()
        wo_copy.wait()

        iq = lax.broadcasted_iota(jnp.int32, (SQ, SKV), 0)
        ik = lax.broadcasted_iota(jnp.int32, (SQ, SKV), 1)
        band = jnp.abs(iq - ik) <= WINDOW

        wq_bf = wq_loc[:, :].astype(jnp.bfloat16)
        wo_bf = wo_loc[:, :].astype(jnp.bfloat16)
        x_bf = x_ref[:, :].astype(jnp.bfloat16)
        q_all = jax.lax.dot(x_bf, wq_bf,
                            preferred_element_type=jnp.float32)

        for b in range(B):
            ctx_cols = []
            for h in range(HQ_LOC):
                qh = q_all[b * SQ:(b + 1) * SQ,
                           h * DH:(h + 1) * DH].astype(jnp.bfloat16)
                kh = k_ref[b, :, h, :].astype(jnp.bfloat16)
                scores = jax.lax.dot_general(
                    qh, kh, (((1,), (1,)), ((), ())),
                    preferred_element_type=jnp.float32) * 0.125
                scores = jnp.where(band, scores, jnp.float32(-1e9))
                m = jnp.max(scores, axis=1, keepdims=True)
                w = jnp.exp(scores - m)
                w = w / jnp.sum(w, axis=1, keepdims=True)
                vh = v_ref[b, :, h, :].astype(jnp.bfloat16)
                ctx_cols.append(jax.lax.dot(
                    w.astype(jnp.bfloat16), vh,
                    preferred_element_type=jnp.float32))
            ctx_b = jnp.concatenate(ctx_cols, axis=1)
            partial_ref[b * SQ:(b + 1) * SQ, :] = jax.lax.dot(
                ctx_b.astype(jnp.bfloat16), wo_bf,
                preferred_element_type=jnp.float32)

        for s in range(N_DEV - 1):
            send_c = _mod(my - s)
            if s == 0:
                src = partial_ref.at[pl.ds(send_c * CHUNK, CHUNK), :]
            else:
                src = rs_buf.at[s - 1]
            rdma = pltpu.make_async_remote_copy(
                src_ref=src,
                dst_ref=rs_buf.at[s],
                send_sem=send_sems.at[s],
                recv_sem=recv_sems.at[s],
                device_id=(right,),
                device_id_type=pl.DeviceIdType.MESH,
            )
            rdma.start()
            rdma.wait()
            recv_c = _mod(my - s - 1)
            rs_buf[s, :, :] = (
                rs_buf[s, :, :]
                + partial_ref[pl.ds(recv_c * CHUNK, CHUNK), :])

        my_c = _mod(my + 1)
        out_ref[pl.ds(my_c * CHUNK, CHUNK), :] = rs_buf[N_DEV - 2, :, :]

        for s in range(N_DEV - 1):
            if s == 0:
                src = rs_buf.at[N_DEV - 2]
            else:
                src = ag_buf.at[s - 1]
            rdma = pltpu.make_async_remote_copy(
                src_ref=src,
                dst_ref=ag_buf.at[s],
                send_sem=send_sems.at[N_DEV - 1 + s],
                recv_sem=recv_sems.at[N_DEV - 1 + s],
                device_id=(right,),
                device_id_type=pl.DeviceIdType.MESH,
            )
            rdma.start()
            rdma.wait()
            c = _mod(my - s)
            out_ref[pl.ds(c * CHUNK, CHUNK), :] = ag_buf[s, :, :]

        @functools.partial(pl.run_scoped,
                           second_barrier=pltpu.SemaphoreType.REGULAR)
        def _(second_barrier):
            for nbr in (left, right):
                pl.semaphore_signal(second_barrier, inc=1, device_id=(nbr,),
                                    device_id_type=pl.DeviceIdType.MESH)
            pl.semaphore_wait(second_barrier, 2)

    out = pl.pallas_call(
        body,
        out_shape=jax.ShapeDtypeStruct((ROWS, D_MODEL), jnp.float32),
        in_specs=[
            pl.BlockSpec(memory_space=pltpu.VMEM),
            pl.BlockSpec(memory_space=pltpu.MemorySpace.HBM),
            pl.BlockSpec(memory_space=pltpu.VMEM),
            pl.BlockSpec(memory_space=pltpu.VMEM),
            pl.BlockSpec(memory_space=pltpu.MemorySpace.HBM),
        ],
        out_specs=pl.BlockSpec(memory_space=pltpu.VMEM),
        scratch_shapes=[
            pltpu.VMEM((D_MODEL, D_HID_LOC), jnp.float32),
            pltpu.VMEM((D_HID_LOC, D_MODEL), jnp.float32),
            pltpu.VMEM((ROWS, D_MODEL), jnp.float32),
            pltpu.VMEM((N_DEV - 1, CHUNK, D_MODEL), jnp.float32),
            pltpu.VMEM((N_DEV - 1, CHUNK, D_MODEL), jnp.float32),
            pltpu.SemaphoreType.DMA((2,)),
            pltpu.SemaphoreType.DMA((2 * (N_DEV - 1),)),
            pltpu.SemaphoreType.DMA((2 * (N_DEV - 1),)),
        ],
        compiler_params=pltpu.CompilerParams(collective_id=0),
    )(x2, Wq, K_ext, V_ext, Wo)

    return out.reshape(B, SQ, D_MODEL)


# device time: 60916 ns/iter; 1.8696x vs baseline; 1.8696x over previous
import functools

import jax
import jax.numpy as jnp
from jax import lax
from jax.experimental import pallas as pl
from jax.experimental.pallas import tpu as pltpu

N_DEV = 8
B = 2
SQ = 512
SKV = 512
HQ_LOC = 8
DH = 64
D_MODEL = 768
D_HID_LOC = HQ_LOC * DH
WINDOW = 128
ROWS = B * SQ
CHUNK = ROWS // N_DEV

G = 3
W = D_MODEL // G
ORDERS = ((4, 2, 1), (2, 1, 4), (1, 4, 2))
RS_BASE = (0, 4, 6)
AG_BASE = (0, 1, 3)
SLOTS = 7


def _ring(tt):
    return tt ^ ((tt >> 1) & 1)


def kernel(x, Wq, K_ext, V_ext, Wo):
    x2 = x.reshape(ROWS, D_MODEL)

    def body(x_ref, wq_ref, k_ref, v_ref, wo_ref, out_ref,
             wq_loc, wo_loc, partial_ref, rs_rbuf,
             copy_sems, rs_send, rs_recv, ag_send, ag_recv):
        my = lax.axis_index("i")
        t = my ^ ((my >> 1) & 1)
        partners = [_ring(t ^ m) for m in (1, 2, 4)]

        wq_copy = pltpu.make_async_copy(
            wq_ref.at[:, pl.ds(my * D_HID_LOC, D_HID_LOC)],
            wq_loc, copy_sems.at[0])
        wq_copy.start()
        wo_copy = pltpu.make_async_copy(
            wo_ref.at[pl.ds(my * D_HID_LOC, D_HID_LOC), :],
            wo_loc, copy_sems.at[1])
        wo_copy.start()

        barrier_sem = pltpu.get_barrier_semaphore()
        for nbr in partners:
            pl.semaphore_signal(barrier_sem, inc=1, device_id=(nbr,),
                                device_id_type=pl.DeviceIdType.MESH)
        pl.semaphore_wait(barrier_sem, len(partners))

        wq_copy.wait()
        wo_copy.wait()

        iq = lax.broadcasted_iota(jnp.int32, (SQ, SKV), 0)
        ik = lax.broadcasted_iota(jnp.int32, (SQ, SKV), 1)
        band = jnp.abs(iq - ik) <= WINDOW

        wq_bf = wq_loc[:, :].astype(jnp.bfloat16)
        wo_bf = wo_loc[:, :].astype(jnp.bfloat16)
        x_bf = x_ref[:, :].astype(jnp.bfloat16)
        q_all = jax.lax.dot(x_bf, wq_bf,
                            preferred_element_type=jnp.float32)

        for b in range(B):
            ctx_cols = []
            for h in range(HQ_LOC):
                qh = q_all[b * SQ:(b + 1) * SQ,
                           h * DH:(h + 1) * DH].astype(jnp.bfloat16)
                kh = k_ref[b, :, h, :].astype(jnp.bfloat16)
                scores = jax.lax.dot_general(
                    qh, kh, (((1,), (1,)), ((), ())),
                    preferred_element_type=jnp.float32) * 0.125
                scores = jnp.where(band, scores, jnp.float32(-1e9))
                m = jnp.max(scores, axis=1, keepdims=True)
                w = jnp.exp(scores - m)
                w = w / jnp.sum(w, axis=1, keepdims=True)
                vh = v_ref[b, :, h, :].astype(jnp.bfloat16)
                ctx_cols.append(jax.lax.dot(
                    w.astype(jnp.bfloat16), vh,
                    preferred_element_type=jnp.float32))
            ctx_b = jnp.concatenate(ctx_cols, axis=1)
            partial_ref[b * SQ:(b + 1) * SQ, :] = jax.lax.dot(
                ctx_b.astype(jnp.bfloat16), wo_bf,
                preferred_element_type=jnp.float32)

        for k in range(3):
            descs = []
            for g in range(G):
                masks = ORDERS[g]
                m = masks[k]
                free = masks[k + 1:]
                q_p = _ring(t ^ m)
                for j in range(4 >> k):
                    f = 0
                    if j & 1:
                        f ^= free[0]
                    if j & 2:
                        f ^= free[1]
                    c_send = t ^ (m ^ f)
                    slot = g * SLOTS + RS_BASE[k] + j
                    rdma = pltpu.make_async_remote_copy(
                        src_ref=partial_ref.at[pl.ds(c_send * CHUNK, CHUNK),
                                               pl.ds(g * W, W)],
                        dst_ref=rs_rbuf.at[slot],
                        send_sem=rs_send.at[slot],
                        recv_sem=rs_recv.at[slot],
                        device_id=(q_p,),
                        device_id_type=pl.DeviceIdType.MESH,
                    )
                    rdma.start()
                    descs.append((rdma, slot, t ^ f, g))
            for rdma, slot, c_recv, g in descs:
                rdma.wait()
                partial_ref[pl.ds(c_recv * CHUNK, CHUNK),
                            pl.ds(g * W, W)] = (
                    partial_ref[pl.ds(c_recv * CHUNK, CHUNK),
                                pl.ds(g * W, W)]
                    + rs_rbuf[slot, :, :])

        out_ref[pl.ds(t * CHUNK, CHUNK), :] = (
            partial_ref[pl.ds(t * CHUNK, CHUNK), :])

        for k in range(3):
            started = []
            waits = []
            for g in range(G):
                rmasks = ORDERS[g][::-1]
                m = rmasks[k]
                q_p = _ring(t ^ m)
                for j in range(1 << k):
                    f = 0
                    if j & 1:
                        f ^= rmasks[0]
                    if j & 2:
                        f ^= rmasks[1]
                    slot = g * SLOTS + AG_BASE[k] + j
                    c_send = t ^ f
                    c_recv = t ^ (m ^ f)
                    send = pltpu.make_async_remote_copy(
                        src_ref=out_ref.at[pl.ds(c_send * CHUNK, CHUNK),
                                           pl.ds(g * W, W)],
                        dst_ref=out_ref.at[pl.ds(c_send * CHUNK, CHUNK),
                                           pl.ds(g * W, W)],
                        send_sem=ag_send.at[slot],
                        recv_sem=ag_recv.at[slot],
                        device_id=(q_p,),
                        device_id_type=pl.DeviceIdType.MESH,
                    )
                    send.start()
                    started.append(send)
                    recv = pltpu.make_async_remote_copy(
                        src_ref=out_ref.at[pl.ds(c_recv * CHUNK, CHUNK),
                                           pl.ds(g * W, W)],
                        dst_ref=out_ref.at[pl.ds(c_recv * CHUNK, CHUNK),
                                           pl.ds(g * W, W)],
                        send_sem=ag_send.at[slot],
                        recv_sem=ag_recv.at[slot],
                        device_id=(q_p,),
                        device_id_type=pl.DeviceIdType.MESH,
                    )
                    waits.append(recv)
            for r in waits:
                r.wait_recv()
            for s in started:
                s.wait_send()

        @functools.partial(pl.run_scoped,
                           second_barrier=pltpu.SemaphoreType.REGULAR)
        def _(second_barrier):
            for nbr in partners:
                pl.semaphore_signal(second_barrier, inc=1, device_id=(nbr,),
                                    device_id_type=pl.DeviceIdType.MESH)
            pl.semaphore_wait(second_barrier, len(partners))

    out = pl.pallas_call(
        body,
        out_shape=jax.ShapeDtypeStruct((ROWS, D_MODEL), jnp.float32),
        in_specs=[
            pl.BlockSpec(memory_space=pltpu.VMEM),
            pl.BlockSpec(memory_space=pltpu.MemorySpace.HBM),
            pl.BlockSpec(memory_space=pltpu.VMEM),
            pl.BlockSpec(memory_space=pltpu.VMEM),
            pl.BlockSpec(memory_space=pltpu.MemorySpace.HBM),
        ],
        out_specs=pl.BlockSpec(memory_space=pltpu.VMEM),
        scratch_shapes=[
            pltpu.VMEM((D_MODEL, D_HID_LOC), jnp.float32),
            pltpu.VMEM((D_HID_LOC, D_MODEL), jnp.float32),
            pltpu.VMEM((ROWS, D_MODEL), jnp.float32),
            pltpu.VMEM((G * SLOTS, CHUNK, W), jnp.float32),
            pltpu.SemaphoreType.DMA((2,)),
            pltpu.SemaphoreType.DMA((G * SLOTS,)),
            pltpu.SemaphoreType.DMA((G * SLOTS,)),
            pltpu.SemaphoreType.DMA((G * SLOTS,)),
            pltpu.SemaphoreType.DMA((G * SLOTS,)),
        ],
        compiler_params=pltpu.CompilerParams(collective_id=0),
    )(x2, Wq, K_ext, V_ext, Wo)

    return out.reshape(B, SQ, D_MODEL)


# device time: 27045 ns/iter; 4.2110x vs baseline; 2.2524x over previous
import functools
import os

import jax
import jax.numpy as jnp
from jax import lax
from jax.experimental import pallas as pl
from jax.experimental.pallas import tpu as pltpu

N_DEV = 8
B = 2
SQ = 512
SKV = 512
HQ_LOC = 8
DH = 64
D_MODEL = 768
D_HID_LOC = HQ_LOC * DH
WINDOW = 128
ROWS = B * SQ
CHUNK = ROWS // N_DEV

G = 3
W = D_MODEL // G
ORDERS = ((4, 2, 1), (2, 1, 4), (1, 4, 2))
RS_BASE = (0, 4, 6)
AG_BASE = (0, 1, 3)
SLOTS = 7

SKIP_COMM = os.environ.get("SKIP_COMM") == "1"
SKIP_COMPUTE = os.environ.get("SKIP_COMPUTE") == "1"


def _ring(tt):
    return tt ^ ((tt >> 1) & 1)


def _compute(x_ref, k_ref, v_ref, wq_loc, wo_loc, partial_ref):
    iq = lax.broadcasted_iota(jnp.int32, (SQ, SKV), 0)
    ik = lax.broadcasted_iota(jnp.int32, (SQ, SKV), 1)
    band = jnp.abs(iq - ik) <= WINDOW

    wq_bf = wq_loc[:, :].astype(jnp.bfloat16)
    wo_bf = wo_loc[:, :].astype(jnp.bfloat16)
    x_bf = x_ref[:, :].astype(jnp.bfloat16)
    q_all = jax.lax.dot(x_bf, wq_bf,
                        preferred_element_type=jnp.float32)

    for b in range(B):
        ctx_cols = []
        for h in range(HQ_LOC):
            qh = q_all[b * SQ:(b + 1) * SQ,
                       h * DH:(h + 1) * DH].astype(jnp.bfloat16)
            kh = k_ref[b, :, h, :].astype(jnp.bfloat16)
            scores = jax.lax.dot_general(
                qh, kh, (((1,), (1,)), ((), ())),
                preferred_element_type=jnp.float32) * 0.125
            scores = jnp.where(band, scores, jnp.float32(-1e9))
            m = jnp.max(scores, axis=1, keepdims=True)
            w = jnp.exp(scores - m)
            w = w / jnp.sum(w, axis=1, keepdims=True)
            vh = v_ref[b, :, h, :].astype(jnp.bfloat16)
            ctx_cols.append(jax.lax.dot(
                w.astype(jnp.bfloat16), vh,
                preferred_element_type=jnp.float32))
        ctx_b = jnp.concatenate(ctx_cols, axis=1)
        partial_ref[b * SQ:(b + 1) * SQ, :] = jax.lax.dot(
            ctx_b.astype(jnp.bfloat16), wo_bf,
            preferred_element_type=jnp.float32)


def kernel(x, Wq, K_ext, V_ext, Wo):
    x2 = x.reshape(ROWS, D_MODEL)

    def body(x_ref, wq_ref, k_ref, v_ref, wo_ref, out_ref,
             wq_loc, wo_loc, partial_ref, rs_rbuf,
             copy_sems, rs_send, rs_recv, ag_send, ag_recv):
        my = lax.axis_index("i")
        t = my ^ ((my >> 1) & 1)
        partners = [_ring(t ^ m) for m in (1, 2, 4)]

        wq_copy = pltpu.make_async_copy(
            wq_ref.at[:, pl.ds(my * D_HID_LOC, D_HID_LOC)],
            wq_loc, copy_sems.at[0])
        wq_copy.start()
        wo_copy = pltpu.make_async_copy(
            wo_ref.at[pl.ds(my * D_HID_LOC, D_HID_LOC), :],
            wo_loc, copy_sems.at[1])
        wo_copy.start()

        if not SKIP_COMM:
            barrier_sem = pltpu.get_barrier_semaphore()
            for nbr in partners:
                pl.semaphore_signal(barrier_sem, inc=1, device_id=(nbr,),
                                    device_id_type=pl.DeviceIdType.MESH)
            pl.semaphore_wait(barrier_sem, len(partners))

        wq_copy.wait()
        wo_copy.wait()

        if not SKIP_COMPUTE:
            _compute(x_ref, k_ref, v_ref, wq_loc, wo_loc, partial_ref)

        if SKIP_COMM:
            out_ref[:, :] = partial_ref[:, :]
            return

        for k in range(3):
            descs = []
            for g in range(G):
                masks = ORDERS[g]
                m = masks[k]
                free = masks[k + 1:]
                q_p = _ring(t ^ m)
                for j in range(4 >> k):
                    f = 0
                    if j & 1:
                        f ^= free[0]
                    if j & 2:
                        f ^= free[1]
                    c_send = t ^ (m ^ f)
                    slot = g * SLOTS + RS_BASE[k] + j
                    rdma = pltpu.make_async_remote_copy(
                        src_ref=partial_ref.at[pl.ds(c_send * CHUNK, CHUNK),
                                               pl.ds(g * W, W)],
                        dst_ref=rs_rbuf.at[slot],
                        send_sem=rs_send.at[slot],
                        recv_sem=rs_recv.at[slot],
                        device_id=(q_p,),
                        device_id_type=pl.DeviceIdType.MESH,
                    )
                    rdma.start()
                    descs.append((rdma, slot, t ^ f, g))
            for rdma, slot, c_recv, g in descs:
                rdma.wait()
                partial_ref[pl.ds(c_recv * CHUNK, CHUNK),
                            pl.ds(g * W, W)] = (
                    partial_ref[pl.ds(c_recv * CHUNK, CHUNK),
                                pl.ds(g * W, W)]
                    + rs_rbuf[slot, :, :])

        out_ref[pl.ds(t * CHUNK, CHUNK), :] = (
            partial_ref[pl.ds(t * CHUNK, CHUNK), :])

        for k in range(3):
            started = []
            waits = []
            for g in range(G):
                rmasks = ORDERS[g][::-1]
                m = rmasks[k]
                q_p = _ring(t ^ m)
                for j in range(1 << k):
                    f = 0
                    if j & 1:
                        f ^= rmasks[0]
                    if j & 2:
                        f ^= rmasks[1]
                    slot = g * SLOTS + AG_BASE[k] + j
                    c_send = t ^ f
                    c_recv = t ^ (m ^ f)
                    send = pltpu.make_async_remote_copy(
                        src_ref=out_ref.at[pl.ds(c_send * CHUNK, CHUNK),
                                           pl.ds(g * W, W)],
                        dst_ref=out_ref.at[pl.ds(c_send * CHUNK, CHUNK),
                                           pl.ds(g * W, W)],
                        send_sem=ag_send.at[slot],
                        recv_sem=ag_recv.at[slot],
                        device_id=(q_p,),
                        device_id_type=pl.DeviceIdType.MESH,
                    )
                    send.start()
                    started.append(send)
                    recv = pltpu.make_async_remote_copy(
                        src_ref=out_ref.at[pl.ds(c_recv * CHUNK, CHUNK),
                                           pl.ds(g * W, W)],
                        dst_ref=out_ref.at[pl.ds(c_recv * CHUNK, CHUNK),
                                           pl.ds(g * W, W)],
                        send_sem=ag_send.at[slot],
                        recv_sem=ag_recv.at[slot],
                        device_id=(q_p,),
                        device_id_type=pl.DeviceIdType.MESH,
                    )
                    waits.append(recv)
            for r in waits:
                r.wait_recv()
            for s in started:
                s.wait_send()

        @functools.partial(pl.run_scoped,
                           second_barrier=pltpu.SemaphoreType.REGULAR)
        def _(second_barrier):
            for nbr in partners:
                pl.semaphore_signal(second_barrier, inc=1, device_id=(nbr,),
                                    device_id_type=pl.DeviceIdType.MESH)
            pl.semaphore_wait(second_barrier, len(partners))

    out = pl.pallas_call(
        body,
        out_shape=jax.ShapeDtypeStruct((ROWS, D_MODEL), jnp.float32),
        in_specs=[
            pl.BlockSpec(memory_space=pltpu.VMEM),
            pl.BlockSpec(memory_space=pltpu.MemorySpace.HBM),
            pl.BlockSpec(memory_space=pltpu.VMEM),
            pl.BlockSpec(memory_space=pltpu.VMEM),
            pl.BlockSpec(memory_space=pltpu.MemorySpace.HBM),
        ],
        out_specs=pl.BlockSpec(memory_space=pltpu.VMEM),
        scratch_shapes=[
            pltpu.VMEM((D_MODEL, D_HID_LOC), jnp.float32),
            pltpu.VMEM((D_HID_LOC, D_MODEL), jnp.float32),
            pltpu.VMEM((ROWS, D_MODEL), jnp.float32),
            pltpu.VMEM((G * SLOTS, CHUNK, W), jnp.float32),
            pltpu.SemaphoreType.DMA((2,)),
            pltpu.SemaphoreType.DMA((G * SLOTS,)),
            pltpu.SemaphoreType.DMA((G * SLOTS,)),
            pltpu.SemaphoreType.DMA((G * SLOTS,)),
            pltpu.SemaphoreType.DMA((G * SLOTS,)),
        ],
        compiler_params=pltpu.CompilerParams(
            collective_id=None if SKIP_COMM else 0),
    )(x2, Wq, K_ext, V_ext, Wo)

    return out.reshape(B, SQ, D_MODEL)
